# baseline (speedup 1.0000x reference)
import functools

import jax
import jax.numpy as jnp
from jax import lax
from jax.experimental import pallas as pl
from jax.experimental.pallas import tpu as pltpu
from jax.experimental.pallas import tpu_sc as plsc

N = 10000
NP = 10240
E = 320000
F = 128
NUM_FEAT = 128
NUM_CLASS = 10

NC = 2
NS = 16
CHUNK = 32
BLKCH = 16
NBLK = 20
NCHUNKS = NBLK * BLKCH
EP = NC * NS * NCHUNKS * CHUNK
ROWS_PER_TILE = NP // NS
NBUF = 8
LAG = NBUF - 1


@functools.cache
def _make_sc_segment_sum():
    mesh = plsc.VectorSubcoreMesh(
        core_axis_name="c", subcore_axis_name="s", num_cores=NC, num_subcores=NS
    )

    @functools.partial(
        pl.kernel,
        out_type=jax.ShapeDtypeStruct((NC, NP, F), jnp.float32),
        mesh=mesh,
        scratch_types=[
            pltpu.VMEM((BLKCH, CHUNK), jnp.int32),
            pltpu.VMEM((BLKCH, CHUNK), jnp.int32),
            pltpu.VMEM((BLKCH, CHUNK), jnp.int32),
            pltpu.VMEM((BLKCH, CHUNK), jnp.int32),
            [pltpu.VMEM((CHUNK, F), jnp.float32)] * NBUF,
            pltpu.VMEM_SHARED((NP, F), jnp.float32),
            [pltpu.SemaphoreType.DMA] * NBUF,
            [pltpu.SemaphoreType.DMA] * NBUF,
            pltpu.SemaphoreType.DMA,
            pltpu.SemaphoreType.DMA,
        ],
    )
    def sc_segment_sum(emb_hbm, src_hbm, dst_hbm, out_hbm,
                       srcA, dstA, srcB, dstB, bufs, acc_sh,
                       gs, ss, isA, isB):
        cid = lax.axis_index("c")
        sid = lax.axis_index("s")
        row0 = sid * ROWS_PER_TILE
        pltpu.sync_copy(emb_hbm.at[pl.ds(row0, ROWS_PER_TILE)],
                        acc_sh.at[pl.ds(row0, ROWS_PER_TILE)])
        plsc.subcore_barrier()

        def idx_load(b, sbuf, dbuf, sem):
            pltpu.async_copy(src_hbm.at[cid, sid, b], sbuf, sem)
            pltpu.async_copy(dst_hbm.at[cid, sid, b], dbuf, sem)

        def idx_wait(sbuf, dbuf, sem):
            pltpu.make_async_copy(src_hbm.at[cid, sid, 0], sbuf, sem).wait()
            pltpu.make_async_copy(dst_hbm.at[cid, sid, 0], dbuf, sem).wait()

        def wait1(buf, sem):
            pltpu.make_async_copy(emb_hbm.at[srcA.at[0]], buf, sem).wait()

        def scat(dref, p):
            wait1(bufs[p], gs[p])
            pltpu.async_copy(bufs[p], acc_sh.at[dref], ss[p], add=True)

        def chunk_op(sb, k, db_cur, db_prev, skip_scatter=False):
            p = k % NBUF
            wait1(bufs[p], ss[p])
            pltpu.async_copy(emb_hbm.at[sb.at[k]], bufs[p], gs[p])
            if not skip_scatter:
                m = k - LAG
                pm = m % NBUF
                scat(db_cur.at[m] if m >= 0 else db_prev.at[BLKCH + m], pm)

        idx_load(0, srcA, dstA, isA)
        idx_wait(srcA, dstA, isA)
        idx_load(1, srcB, dstB, isB)
        for p in range(NBUF):
            pltpu.async_copy(emb_hbm.at[pl.ds(row0, CHUNK)], bufs[p], ss[p])
        for k in range(BLKCH):
            chunk_op(srcA, k, dstA, None, skip_scatter=(k < LAG))

        @pl.loop(1, NBLK - 1, step=2)
        def _block_pair(b):
            idx_wait(srcB, dstB, isB)
            for k in range(BLKCH):
                chunk_op(srcB, k, dstB, dstA)
                if k == NBUF:
                    idx_load(b + 1, srcA, dstA, isA)
            idx_wait(srcA, dstA, isA)
            for k in range(BLKCH):
                chunk_op(srcA, k, dstA, dstB)
                if k == NBUF:
                    idx_load(b + 2, srcB, dstB, isB)

        idx_wait(srcB, dstB, isB)
        for k in range(BLKCH):
            chunk_op(srcB, k, dstB, dstA)
        for k in range(BLKCH - LAG, BLKCH):
            scat(dstB.at[k], k % NBUF)
        for p in range(NBUF):
            wait1(bufs[p], ss[p])

        plsc.subcore_barrier()
        pltpu.sync_copy(acc_sh.at[pl.ds(row0, ROWS_PER_TILE)],
                        out_hbm.at[cid, pl.ds(row0, ROWS_PER_TILE)])

    return sc_segment_sum


def _sc_segment_sum(emb, src, dst):
    return _make_sc_segment_sum()(emb, src, dst)


_EMB_BLK = 1024


def _emb_body(ids_ref, table_ref, out_ref):
    ids = ids_ref[...]
    oh = (ids == lax.broadcasted_iota(jnp.int32, (_EMB_BLK, NUM_FEAT), 1))
    out_ref[...] = jnp.dot(oh.astype(jnp.float32), table_ref[...],
                           preferred_element_type=jnp.float32)


def _embed(node_feature, emb_table):
    ids = jnp.full((NP, 1), NUM_FEAT, dtype=jnp.int32)
    ids = ids.at[:N, 0].set(node_feature.astype(jnp.int32))
    return pl.pallas_call(
        _emb_body,
        grid=(NP // _EMB_BLK,),
        in_specs=[
            pl.BlockSpec((_EMB_BLK, 1), lambda i: (i, 0)),
            pl.BlockSpec((NUM_FEAT, F), lambda i: (0, 0)),
        ],
        out_specs=pl.BlockSpec((_EMB_BLK, F), lambda i: (i, 0)),
        out_shape=jax.ShapeDtypeStruct((NP, F), jnp.float32),
    )(ids, emb_table)


_DENSE_BLK = 1024


def _dense_body(p0_ref, p1_ref, emb_ref, wh_ref, bh_ref, wfp_ref, bfp_ref,
                fin_ref, wcl_ref, bcl_ref, r_ref, f_ref, out_ref):
    i = pl.program_id(0)
    row = i * _DENSE_BLK + lax.broadcasted_iota(jnp.int32, (_DENSE_BLK, 1), 0)
    valid = (row < N).astype(jnp.float32)
    v = p0_ref[...] + p1_ref[...] - emb_ref[...]
    h = jnp.dot(v, wh_ref[...], preferred_element_type=jnp.float32) + bh_ref[...]
    h = jnp.maximum(h, 0.0) * valid
    r_ref[...] = h
    s = jnp.dot(h, wfp_ref[...], preferred_element_type=jnp.float32) + bfp_ref[...]
    s = s - jnp.max(s, axis=-1, keepdims=True)
    e = jnp.exp(s)
    sm = e / jnp.sum(e, axis=-1, keepdims=True)

    @pl.when(i == 0)
    def _():
        f_ref[...] = fin_ref[...]

    f_ref[...] += jnp.sum(sm * valid, axis=0, keepdims=True)

    @pl.when(i == NP // _DENSE_BLK - 1)
    def _():
        c = jnp.dot(f_ref[...], wcl_ref[...],
                    preferred_element_type=jnp.float32) + bcl_ref[...]
        c = c - jnp.max(c, axis=-1, keepdims=True)
        out_ref[...] = c - jnp.log(jnp.sum(jnp.exp(c), axis=-1, keepdims=True))


def _dense_round(p0, p1, emb, wh, bh, wfp, bfp, f_in, wcl, bcl):
    return pl.pallas_call(
        _dense_body,
        grid=(NP // _DENSE_BLK,),
        in_specs=[
            pl.BlockSpec((_DENSE_BLK, F), lambda i: (i, 0)),
            pl.BlockSpec((_DENSE_BLK, F), lambda i: (i, 0)),
            pl.BlockSpec((_DENSE_BLK, F), lambda i: (i, 0)),
            pl.BlockSpec((F, F), lambda i: (0, 0)),
            pl.BlockSpec((1, F), lambda i: (0, 0)),
            pl.BlockSpec((F, F), lambda i: (0, 0)),
            pl.BlockSpec((1, F), lambda i: (0, 0)),
            pl.BlockSpec((1, F), lambda i: (0, 0)),
            pl.BlockSpec((F, NUM_CLASS), lambda i: (0, 0)),
            pl.BlockSpec((1, NUM_CLASS), lambda i: (0, 0)),
        ],
        out_specs=[
            pl.BlockSpec((_DENSE_BLK, F), lambda i: (i, 0)),
            pl.BlockSpec((1, F), lambda i: (0, 0)),
            pl.BlockSpec((1, NUM_CLASS), lambda i: (0, 0)),
        ],
        out_shape=[
            jax.ShapeDtypeStruct((NP, F), jnp.float32),
            jax.ShapeDtypeStruct((1, F), jnp.float32),
            jax.ShapeDtypeStruct((1, NUM_CLASS), jnp.float32),
        ],
    )(p0, p1, emb, wh, bh.reshape(1, F), wfp, bfp.reshape(1, F),
      f_in, wcl, bcl.reshape(1, NUM_CLASS))


def kernel(node_feature, edge_index, emb_table, Wh, bh, Wfp, bfp, Wcl, bcl):
    pad = jnp.arange(EP - E, dtype=jnp.int32)
    src = jnp.concatenate([edge_index[0].astype(jnp.int32), pad % N])
    dst = jnp.concatenate([edge_index[1].astype(jnp.int32), N + pad % (NP - N)])
    src = src.reshape(NC, NS, NBLK, BLKCH, CHUNK)
    dst = dst.reshape(NC, NS, NBLK, BLKCH, CHUNK)
    emb = _embed(node_feature, emb_table)
    f = jnp.zeros((1, F), dtype=jnp.float32)
    logits = None
    for l in range(3):
        p = _sc_segment_sum(emb, src, dst)
        emb, f, logits = _dense_round(p[0], p[1], emb, Wh[l], bh[l],
                                      Wfp[l], bfp[l], f, Wcl, bcl)
    return logits.reshape(NUM_CLASS)


# speedup vs baseline: 1.0131x; 1.0131x over previous
import functools

import jax
import jax.numpy as jnp
from jax import lax
from jax.experimental import pallas as pl
from jax.experimental.pallas import tpu as pltpu
from jax.experimental.pallas import tpu_sc as plsc

N = 10000
NP = 10240
E = 320000
F = 128
NUM_FEAT = 128
NUM_CLASS = 10

NC = 2
NS = 16
CHUNK = 80
BLKCH = 8
NBLK = 16
NCHUNKS = NBLK * BLKCH
EP = NC * NS * NCHUNKS * CHUNK
ROWS_PER_TILE = NP // NS
NBUF = 4
LAG = NBUF - 1


@functools.cache
def _make_sc_segment_sum():
    mesh = plsc.VectorSubcoreMesh(
        core_axis_name="c", subcore_axis_name="s", num_cores=NC, num_subcores=NS
    )

    @functools.partial(
        pl.kernel,
        out_type=jax.ShapeDtypeStruct((NC, NP, F), jnp.float32),
        mesh=mesh,
        scratch_types=[
            pltpu.VMEM((BLKCH, CHUNK), jnp.int32),
            pltpu.VMEM((BLKCH, CHUNK), jnp.int32),
            pltpu.VMEM((BLKCH, CHUNK), jnp.int32),
            pltpu.VMEM((BLKCH, CHUNK), jnp.int32),
            [pltpu.VMEM((CHUNK, F), jnp.float32)] * NBUF,
            pltpu.VMEM_SHARED((NP, F), jnp.float32),
            [pltpu.SemaphoreType.DMA] * NBUF,
            [pltpu.SemaphoreType.DMA] * NBUF,
            pltpu.SemaphoreType.DMA,
            pltpu.SemaphoreType.DMA,
        ],
    )
    def sc_segment_sum(emb_hbm, src_hbm, dst_hbm, out_hbm,
                       srcA, dstA, srcB, dstB, bufs, acc_sh,
                       gs, ss, isA, isB):
        cid = lax.axis_index("c")
        sid = lax.axis_index("s")
        row0 = sid * ROWS_PER_TILE
        pltpu.sync_copy(emb_hbm.at[pl.ds(row0, ROWS_PER_TILE)],
                        acc_sh.at[pl.ds(row0, ROWS_PER_TILE)])
        plsc.subcore_barrier()

        def idx_load(b, sbuf, dbuf, sem):
            pltpu.async_copy(src_hbm.at[cid, sid, b], sbuf, sem)
            pltpu.async_copy(dst_hbm.at[cid, sid, b], dbuf, sem)

        def idx_wait(sbuf, dbuf, sem):
            pltpu.make_async_copy(src_hbm.at[cid, sid, 0], sbuf, sem).wait()
            pltpu.make_async_copy(dst_hbm.at[cid, sid, 0], dbuf, sem).wait()

        def wait1(buf, sem):
            pltpu.make_async_copy(emb_hbm.at[srcA.at[0]], buf, sem).wait()

        def scat(dref, p):
            wait1(bufs[p], gs[p])
            pltpu.async_copy(bufs[p], acc_sh.at[dref], ss[p], add=True)

        def chunk_op(sb, k, db_cur, db_prev, skip_scatter=False):
            p = k % NBUF
            wait1(bufs[p], ss[p])
            pltpu.async_copy(emb_hbm.at[sb.at[k]], bufs[p], gs[p])
            if not skip_scatter:
                m = k - LAG
                pm = m % NBUF
                scat(db_cur.at[m] if m >= 0 else db_prev.at[BLKCH + m], pm)

        idx_load(0, srcA, dstA, isA)
        idx_wait(srcA, dstA, isA)
        idx_load(1, srcB, dstB, isB)
        for p in range(NBUF):
            pltpu.async_copy(emb_hbm.at[pl.ds(row0, CHUNK)], bufs[p], ss[p])
        for k in range(BLKCH):
            chunk_op(srcA, k, dstA, None, skip_scatter=(k < LAG))

        @pl.loop(1, NBLK - 1, step=2)
        def _block_pair(b):
            idx_wait(srcB, dstB, isB)
            for k in range(BLKCH):
                chunk_op(srcB, k, dstB, dstA)
                if k == NBUF:
                    idx_load(b + 1, srcA, dstA, isA)
            idx_wait(srcA, dstA, isA)
            for k in range(BLKCH):
                chunk_op(srcA, k, dstA, dstB)
                if k == NBUF:
                    idx_load(b + 2, srcB, dstB, isB)

        idx_wait(srcB, dstB, isB)
        for k in range(BLKCH):
            chunk_op(srcB, k, dstB, dstA)
        for k in range(BLKCH - LAG, BLKCH):
            scat(dstB.at[k], k % NBUF)
        for p in range(NBUF):
            wait1(bufs[p], ss[p])

        plsc.subcore_barrier()
        pltpu.sync_copy(acc_sh.at[pl.ds(row0, ROWS_PER_TILE)],
                        out_hbm.at[cid, pl.ds(row0, ROWS_PER_TILE)])

    return sc_segment_sum


def _sc_segment_sum(emb, src, dst):
    return _make_sc_segment_sum()(emb, src, dst)


_EMB_BLK = 1024


def _emb_body(ids_ref, table_ref, out_ref):
    ids = ids_ref[...]
    oh = (ids == lax.broadcasted_iota(jnp.int32, (_EMB_BLK, NUM_FEAT), 1))
    out_ref[...] = jnp.dot(oh.astype(jnp.float32), table_ref[...],
                           preferred_element_type=jnp.float32)


def _embed(node_feature, emb_table):
    ids = jnp.full((NP, 1), NUM_FEAT, dtype=jnp.int32)
    ids = ids.at[:N, 0].set(node_feature.astype(jnp.int32))
    return pl.pallas_call(
        _emb_body,
        grid=(NP // _EMB_BLK,),
        in_specs=[
            pl.BlockSpec((_EMB_BLK, 1), lambda i: (i, 0)),
            pl.BlockSpec((NUM_FEAT, F), lambda i: (0, 0)),
        ],
        out_specs=pl.BlockSpec((_EMB_BLK, F), lambda i: (i, 0)),
        out_shape=jax.ShapeDtypeStruct((NP, F), jnp.float32),
    )(ids, emb_table)


_DENSE_BLK = 1024


def _dense_body(p0_ref, p1_ref, emb_ref, wh_ref, bh_ref, wfp_ref, bfp_ref,
                fin_ref, wcl_ref, bcl_ref, r_ref, f_ref, out_ref):
    i = pl.program_id(0)
    row = i * _DENSE_BLK + lax.broadcasted_iota(jnp.int32, (_DENSE_BLK, 1), 0)
    valid = (row < N).astype(jnp.float32)
    v = p0_ref[...] + p1_ref[...] - emb_ref[...]
    h = jnp.dot(v, wh_ref[...], preferred_element_type=jnp.float32) + bh_ref[...]
    h = jnp.maximum(h, 0.0) * valid
    r_ref[...] = h
    s = jnp.dot(h, wfp_ref[...], preferred_element_type=jnp.float32) + bfp_ref[...]
    s = s - jnp.max(s, axis=-1, keepdims=True)
    e = jnp.exp(s)
    sm = e / jnp.sum(e, axis=-1, keepdims=True)

    @pl.when(i == 0)
    def _():
        f_ref[...] = fin_ref[...]

    f_ref[...] += jnp.sum(sm * valid, axis=0, keepdims=True)

    @pl.when(i == NP // _DENSE_BLK - 1)
    def _():
        c = jnp.dot(f_ref[...], wcl_ref[...],
                    preferred_element_type=jnp.float32) + bcl_ref[...]
        c = c - jnp.max(c, axis=-1, keepdims=True)
        out_ref[...] = c - jnp.log(jnp.sum(jnp.exp(c), axis=-1, keepdims=True))


def _dense_round(p0, p1, emb, wh, bh, wfp, bfp, f_in, wcl, bcl):
    return pl.pallas_call(
        _dense_body,
        grid=(NP // _DENSE_BLK,),
        in_specs=[
            pl.BlockSpec((_DENSE_BLK, F), lambda i: (i, 0)),
            pl.BlockSpec((_DENSE_BLK, F), lambda i: (i, 0)),
            pl.BlockSpec((_DENSE_BLK, F), lambda i: (i, 0)),
            pl.BlockSpec((F, F), lambda i: (0, 0)),
            pl.BlockSpec((1, F), lambda i: (0, 0)),
            pl.BlockSpec((F, F), lambda i: (0, 0)),
            pl.BlockSpec((1, F), lambda i: (0, 0)),
            pl.BlockSpec((1, F), lambda i: (0, 0)),
            pl.BlockSpec((F, NUM_CLASS), lambda i: (0, 0)),
            pl.BlockSpec((1, NUM_CLASS), lambda i: (0, 0)),
        ],
        out_specs=[
            pl.BlockSpec((_DENSE_BLK, F), lambda i: (i, 0)),
            pl.BlockSpec((1, F), lambda i: (0, 0)),
            pl.BlockSpec((1, NUM_CLASS), lambda i: (0, 0)),
        ],
        out_shape=[
            jax.ShapeDtypeStruct((NP, F), jnp.float32),
            jax.ShapeDtypeStruct((1, F), jnp.float32),
            jax.ShapeDtypeStruct((1, NUM_CLASS), jnp.float32),
        ],
    )(p0, p1, emb, wh, bh.reshape(1, F), wfp, bfp.reshape(1, F),
      f_in, wcl, bcl.reshape(1, NUM_CLASS))


def kernel(node_feature, edge_index, emb_table, Wh, bh, Wfp, bfp, Wcl, bcl):
    pad = jnp.arange(EP - E, dtype=jnp.int32)
    src = jnp.concatenate([edge_index[0].astype(jnp.int32), pad % N])
    dst = jnp.concatenate([edge_index[1].astype(jnp.int32), N + pad % (NP - N)])
    src = src.reshape(NC, NS, NBLK, BLKCH, CHUNK)
    dst = dst.reshape(NC, NS, NBLK, BLKCH, CHUNK)
    emb = _embed(node_feature, emb_table)
    f = jnp.zeros((1, F), dtype=jnp.float32)
    logits = None
    for l in range(3):
        p = _sc_segment_sum(emb, src, dst)
        emb, f, logits = _dense_round(p[0], p[1], emb, Wh[l], bh[l],
                                      Wfp[l], bfp[l], f, Wcl, bcl)
    return logits.reshape(NUM_CLASS)


# speedup vs baseline: 1.0758x; 1.0618x over previous
import functools

import jax
import jax.numpy as jnp
from jax import lax
from jax.experimental import pallas as pl
from jax.experimental.pallas import tpu as pltpu
from jax.experimental.pallas import tpu_sc as plsc

N = 10000
NP = 10240
E = 320000
F = 128
NUM_FEAT = 128
NUM_CLASS = 10

NC = 2
NS = 16
CHUNK = 64
BLKCH = 8
NBLK = 20
NCHUNKS = NBLK * BLKCH
EP = NC * NS * NCHUNKS * CHUNK
ROWS_PER_TILE = NP // NS
NBUF = 4
LAG = NBUF - 1


@functools.cache
def _make_sc_segment_sum():
    mesh = plsc.VectorSubcoreMesh(
        core_axis_name="c", subcore_axis_name="s", num_cores=NC, num_subcores=NS
    )

    @functools.partial(
        pl.kernel,
        out_type=jax.ShapeDtypeStruct((NC, NP, F), jnp.float32),
        mesh=mesh,
        scratch_types=[
            pltpu.VMEM((BLKCH, CHUNK), jnp.int32),
            pltpu.VMEM((BLKCH, CHUNK), jnp.int32),
            pltpu.VMEM((BLKCH, CHUNK), jnp.int32),
            pltpu.VMEM((BLKCH, CHUNK), jnp.int32),
            [pltpu.VMEM((CHUNK, F), jnp.float32)] * NBUF,
            pltpu.VMEM_SHARED((NP, F), jnp.float32),
            [pltpu.SemaphoreType.DMA] * NBUF,
            [pltpu.SemaphoreType.DMA] * NBUF,
            pltpu.SemaphoreType.DMA,
            pltpu.SemaphoreType.DMA,
        ],
    )
    def sc_segment_sum(emb_hbm, src_hbm, dst_hbm, out_hbm,
                       srcA, dstA, srcB, dstB, bufs, acc_sh,
                       gs, ss, isA, isB):
        cid = lax.axis_index("c")
        sid = lax.axis_index("s")
        row0 = sid * ROWS_PER_TILE
        pltpu.sync_copy(emb_hbm.at[pl.ds(row0, ROWS_PER_TILE)],
                        acc_sh.at[pl.ds(row0, ROWS_PER_TILE)])
        plsc.subcore_barrier()

        def idx_load(b, sbuf, dbuf, sem):
            pltpu.async_copy(src_hbm.at[cid, sid, b], sbuf, sem)
            pltpu.async_copy(dst_hbm.at[cid, sid, b], dbuf, sem)

        def idx_wait(sbuf, dbuf, sem):
            pltpu.make_async_copy(src_hbm.at[cid, sid, 0], sbuf, sem).wait()
            pltpu.make_async_copy(dst_hbm.at[cid, sid, 0], dbuf, sem).wait()

        def wait1(buf, sem):
            pltpu.make_async_copy(emb_hbm.at[srcA.at[0]], buf, sem).wait()

        def scat(dref, p):
            wait1(bufs[p], gs[p])
            pltpu.async_copy(bufs[p], acc_sh.at[dref], ss[p], add=True)

        def chunk_op(sb, k, db_cur, db_prev, skip_scatter=False):
            p = k % NBUF
            wait1(bufs[p], ss[p])
            pltpu.async_copy(emb_hbm.at[sb.at[k]], bufs[p], gs[p])
            if not skip_scatter:
                m = k - LAG
                pm = m % NBUF
                scat(db_cur.at[m] if m >= 0 else db_prev.at[BLKCH + m], pm)

        idx_load(0, srcA, dstA, isA)
        idx_wait(srcA, dstA, isA)
        idx_load(1, srcB, dstB, isB)
        for p in range(NBUF):
            pltpu.async_copy(emb_hbm.at[pl.ds(row0, CHUNK)], bufs[p], ss[p])
        for k in range(BLKCH):
            chunk_op(srcA, k, dstA, None, skip_scatter=(k < LAG))

        @pl.loop(1, NBLK - 1, step=2)
        def _block_pair(b):
            idx_wait(srcB, dstB, isB)
            for k in range(BLKCH):
                chunk_op(srcB, k, dstB, dstA)
                if k == NBUF:
                    idx_load(b + 1, srcA, dstA, isA)
            idx_wait(srcA, dstA, isA)
            for k in range(BLKCH):
                chunk_op(srcA, k, dstA, dstB)
                if k == NBUF:
                    idx_load(b + 2, srcB, dstB, isB)

        idx_wait(srcB, dstB, isB)
        for k in range(BLKCH):
            chunk_op(srcB, k, dstB, dstA)
        for k in range(BLKCH - LAG, BLKCH):
            scat(dstB.at[k], k % NBUF)
        for p in range(NBUF):
            wait1(bufs[p], ss[p])

        plsc.subcore_barrier()
        pltpu.sync_copy(acc_sh.at[pl.ds(row0, ROWS_PER_TILE)],
                        out_hbm.at[cid, pl.ds(row0, ROWS_PER_TILE)])

    return sc_segment_sum


def _sc_segment_sum(emb, src, dst):
    return _make_sc_segment_sum()(emb, src, dst)


_EMB_BLK = 1024


def _emb_body(ids_ref, table_ref, out_ref):
    ids = ids_ref[...]
    oh = (ids == lax.broadcasted_iota(jnp.int32, (_EMB_BLK, NUM_FEAT), 1))
    out_ref[...] = jnp.dot(oh.astype(jnp.float32), table_ref[...],
                           preferred_element_type=jnp.float32)


def _embed(node_feature, emb_table):
    ids = jnp.full((NP, 1), NUM_FEAT, dtype=jnp.int32)
    ids = ids.at[:N, 0].set(node_feature.astype(jnp.int32))
    return pl.pallas_call(
        _emb_body,
        grid=(NP // _EMB_BLK,),
        in_specs=[
            pl.BlockSpec((_EMB_BLK, 1), lambda i: (i, 0)),
            pl.BlockSpec((NUM_FEAT, F), lambda i: (0, 0)),
        ],
        out_specs=pl.BlockSpec((_EMB_BLK, F), lambda i: (i, 0)),
        out_shape=jax.ShapeDtypeStruct((NP, F), jnp.float32),
    )(ids, emb_table)


_DENSE_BLK = 1024


def _dense_body(l, p_ref, emb_ref, wh_ref, bh_ref, wfp_ref, bfp_ref,
                fin_ref, wcl_ref, bcl_ref, r_ref, f_ref, out_ref):
    i = pl.program_id(0)
    row = i * _DENSE_BLK + lax.broadcasted_iota(jnp.int32, (_DENSE_BLK, 1), 0)
    valid = (row < N).astype(jnp.float32)
    v = p_ref[0] + p_ref[1] - emb_ref[...]
    h = jnp.dot(v, wh_ref[0], preferred_element_type=jnp.float32) \
        + bh_ref[...][l:l + 1]
    h = jnp.maximum(h, 0.0) * valid
    r_ref[...] = h
    s = jnp.dot(h, wfp_ref[0], preferred_element_type=jnp.float32) \
        + bfp_ref[...][l:l + 1]
    s = s - jnp.max(s, axis=-1, keepdims=True)
    e = jnp.exp(s)
    sm = e / jnp.sum(e, axis=-1, keepdims=True)

    @pl.when(i == 0)
    def _():
        f_ref[...] = fin_ref[...]

    f_ref[...] += jnp.sum(sm * valid, axis=0, keepdims=True)

    @pl.when(i == NP // _DENSE_BLK - 1)
    def _():
        c = jnp.dot(f_ref[...], wcl_ref[...],
                    preferred_element_type=jnp.float32) + bcl_ref[...]
        c = c - jnp.max(c, axis=-1, keepdims=True)
        out_ref[...] = c - jnp.log(jnp.sum(jnp.exp(c), axis=-1, keepdims=True))


def _dense_round(p, emb, l, Wh, bh, Wfp, bfp, f_in, wcl, bcl):
    return pl.pallas_call(
        functools.partial(_dense_body, l),
        grid=(NP // _DENSE_BLK,),
        in_specs=[
            pl.BlockSpec((NC, _DENSE_BLK, F), lambda i: (0, i, 0)),
            pl.BlockSpec((_DENSE_BLK, F), lambda i: (i, 0)),
            pl.BlockSpec((1, F, F), lambda i: (l, 0, 0)),
            pl.BlockSpec((3, F), lambda i: (0, 0)),
            pl.BlockSpec((1, F, F), lambda i: (l, 0, 0)),
            pl.BlockSpec((3, F), lambda i: (0, 0)),
            pl.BlockSpec((1, F), lambda i: (0, 0)),
            pl.BlockSpec((F, NUM_CLASS), lambda i: (0, 0)),
            pl.BlockSpec((1, NUM_CLASS), lambda i: (0, 0)),
        ],
        out_specs=[
            pl.BlockSpec((_DENSE_BLK, F), lambda i: (i, 0)),
            pl.BlockSpec((1, F), lambda i: (0, 0)),
            pl.BlockSpec((1, NUM_CLASS), lambda i: (0, 0)),
        ],
        out_shape=[
            jax.ShapeDtypeStruct((NP, F), jnp.float32),
            jax.ShapeDtypeStruct((1, F), jnp.float32),
            jax.ShapeDtypeStruct((1, NUM_CLASS), jnp.float32),
        ],
    )(p, emb, Wh, bh, Wfp, bfp, f_in, wcl, bcl.reshape(1, NUM_CLASS))


def kernel(node_feature, edge_index, emb_table, Wh, bh, Wfp, bfp, Wcl, bcl):
    pad = jnp.arange(EP - E, dtype=jnp.int32)
    src = jnp.concatenate([edge_index[0].astype(jnp.int32), pad % N])
    dst = jnp.concatenate([edge_index[1].astype(jnp.int32), N + pad % (NP - N)])
    src = src.reshape(NC, NS, NBLK, BLKCH, CHUNK)
    dst = dst.reshape(NC, NS, NBLK, BLKCH, CHUNK)
    emb = _embed(node_feature, emb_table)
    f = jnp.zeros((1, F), dtype=jnp.float32)
    logits = None
    for l in range(3):
        p = _sc_segment_sum(emb, src, dst)
        emb, f, logits = _dense_round(p, emb, l, Wh, bh, Wfp, bfp, f, Wcl, bcl)
    return logits.reshape(NUM_CLASS)
